# baseline (speedup 1.0000x reference)
import functools

import jax
import jax.numpy as jnp
from jax import lax
from jax.experimental import pallas as pl
from jax.experimental.pallas import tpu as pltpu
from jax.experimental.pallas import tpu_sc as plsc

N_NEIGHBOURS = 40
N_BITS = 16
CAP = 48
BUF = 80


def _bf16r(a):
    bits = lax.bitcast_convert_type(a, jnp.int32)
    r = bits + jnp.int32(0x7FFF) + ((bits >> 16) & 1)
    return lax.bitcast_convert_type(r & jnp.int32(-65536), jnp.float32)


def _dot16(a, b):
    return jnp.dot(_bf16r(a), _bf16r(b), preferred_element_type=jnp.float32,
                   precision=lax.Precision.HIGHEST)


def _dotx(a, b):
    return jnp.dot(a, b, preferred_element_type=jnp.float32,
                   precision=lax.Precision.HIGHEST)


def _stage1_body(coords_ref,
                 dist_ref, thr_ref, m0_ref,
                 cT_s, cTr_s, c2r_s,
                 *, R, V, n_dim, n_prop):
    j = pl.program_id(1)
    K = N_NEIGHBOURS

    @pl.when(j == 0)
    def _precompute():
        coords = coords_ref[0]
        cT = coords.T
        cT_s[...] = cT
        cTr_s[...] = _bf16r(cT)
        c2r_s[...] = jnp.sum(cT * cT, axis=0, keepdims=True)

    rows = pl.ds(j * R, R)
    cb = coords_ref[0, rows, :]
    cbr = _bf16r(cb)
    g = cbr[:, 0:1] * cTr_s[0:1, :]
    for dd in range(1, n_dim):
        g = g + cbr[:, dd:dd + 1] * cTr_s[dd:dd + 1, :]
    c2b = jnp.sum(cb * cb, axis=1, keepdims=True)
    draw = c2b + c2r_s[...] - 2.0 * g
    dist_ref[...] = draw
    bits = lax.bitcast_convert_type(draw, jnp.int32)
    key = jnp.where(bits >= 0, bits, -(bits & jnp.int32(0x7FFFFFFF)))

    def bit_body(i, p):
        b = 30 - i
        c = p | (jnp.int32(1) << b)
        cnt = jnp.sum((key < c).astype(jnp.int32), axis=1, keepdims=True)
        return jnp.where(cnt >= K, p, c)

    p = lax.fori_loop(0, N_BITS, bit_body, jnp.zeros((R, 1), jnp.int32))
    p = p | jnp.int32((1 << (31 - N_BITS)) - 1)
    thr_ref[0] = lax.bitcast_convert_type(p, jnp.float32)
    colg = lax.broadcasted_iota(jnp.int32, (R, V), 1)
    m0val = jnp.min(key, axis=1, keepdims=True)
    m0idx = jnp.min(jnp.where(key == m0val, colg, V), axis=1, keepdims=True)
    m0_ref[0] = m0idx


def _tc_stage1(coords):
    B, V, n_dim = coords.shape
    R = 256
    NBLK = (B * V) // R
    body = functools.partial(_stage1_body, R=R, V=V, n_dim=n_dim,
                             n_prop=0)
    grid = (B, V // R)
    dist, thr, m0 = pl.pallas_call(
        body,
        grid=grid,
        in_specs=[
            pl.BlockSpec((1, V, n_dim), lambda b, j: (b, 0, 0)),
        ],
        out_specs=[
            pl.BlockSpec((R, V), lambda b, j: (b * (V // 256) + j, 0)),
            pl.BlockSpec((1, R, 1), lambda b, j: (b * (V // 256) + j, 0, 0)),
            pl.BlockSpec((1, R, 1), lambda b, j: (b * (V // 256) + j, 0, 0)),
        ],
        out_shape=[
            jax.ShapeDtypeStruct((B * V, V), jnp.float32),
            jax.ShapeDtypeStruct((NBLK, R, 1), jnp.float32),
            jax.ShapeDtypeStruct((NBLK, R, 1), jnp.int32),
        ],
        scratch_shapes=[
            pltpu.VMEM((n_dim, V), jnp.float32),
            pltpu.VMEM((n_dim, V), jnp.float32),
            pltpu.VMEM((1, V), jnp.float32),
        ],
        compiler_params=pltpu.CompilerParams(
            dimension_semantics=("arbitrary", "arbitrary"),
        ),
    )(coords)
    return dist, thr.reshape(B * V), m0.reshape(B * V)


def _sc_aggregate(dist, thr, m0, feats_flat, B, V, P):
    NB = B * V
    NW = 32
    RPW = NB // NW
    FP = V * P
    mesh = plsc.VectorSubcoreMesh(core_axis_name="c", subcore_axis_name="s")

    @functools.partial(
        pl.kernel, mesh=mesh,
        out_type=jax.ShapeDtypeStruct((NB * 48,), jnp.float32),
        scratch_types=[
            pltpu.VMEM((FP,), jnp.float32),
            pltpu.VMEM((2 * V,), jnp.float32),
            pltpu.VMEM((RPW + 16,), jnp.float32),
            pltpu.VMEM((RPW + 16,), jnp.int32),
            pltpu.VMEM((BUF,), jnp.float32),
            pltpu.VMEM((BUF,), jnp.int32),
            pltpu.VMEM((CAP,), jnp.float32),
            pltpu.VMEM((RPW * 48,), jnp.float32),
            pltpu.SemaphoreType.DMA,
            pltpu.SemaphoreType.DMA,
        ],
        compiler_params=pltpu.CompilerParams(needs_layout_passes=False))
    def sck(dist_hbm, thr_hbm, m0_hbm, feats_hbm, out_hbm,
            feats_v, drow_v, thr_v, m0_v, cd_v, ci_v, cw_v, orow_v,
            sem0, sem1):
        wid = lax.axis_index("s") * 2 + lax.axis_index("c")
        row0 = wid * RPW
        batch = row0 // V
        i0 = row0 % V
        pltpu.sync_copy(feats_hbm.at[pl.ds(batch * FP, FP)], feats_v)
        pltpu.sync_copy(thr_hbm.at[pl.ds(row0, RPW)],
                        thr_v.at[pl.ds(0, RPW)])
        pltpu.sync_copy(m0_hbm.at[pl.ds(row0, RPW)],
                        m0_v.at[pl.ds(0, RPW)])
        iota = lax.iota(jnp.int32, 16)
        zero16 = jnp.zeros((16,), jnp.float32)
        sems = (sem0, sem1)
        for k in range(BUF // 16):
            ci_v[pl.ds(k * 16, 16)] = jnp.zeros((16,), jnp.int32)

        pltpu.async_copy(dist_hbm.at[pl.ds(row0 * V, V)],
                         drow_v.at[pl.ds(0, V)], sem0)

        def do_row(g, par):
            base0 = par * V
            pltpu.make_async_copy(dist_hbm.at[pl.ds((row0 + g) * V, V)],
                                  drow_v.at[pl.ds(base0, V)],
                                  sems[par]).wait()
            nxt = jnp.minimum(row0 + g + 1, NB - 1)
            pltpu.async_copy(dist_hbm.at[pl.ds(nxt * V, V)],
                             drow_v.at[pl.ds((1 - par) * V, V)],
                             sems[1 - par])
            t = thr_v[pl.ds(g, 16)][0]
            i_self = m0_v[pl.ds(g, 16)][0]

            def scan_body(cc, n):
                for u in range(4):
                    cbase = cc * 64 + u * 16
                    dv = drow_v[pl.ds(base0 + cbase, 16)]
                    jv = iota + cbase
                    m = (dv <= t) & (jv != i_self) & (n < CAP)
                    plsc.store_compressed(cd_v.at[pl.ds(n, 16)], dv, mask=m)
                    plsc.store_compressed(ci_v.at[pl.ds(n, 16)], jv, mask=m)
                    pc = plsc.all_reduce_population_count(m)
                    n = n + pc[0]
                return n

            n = lax.fori_loop(0, V // 64, scan_body, jnp.int32(0))

            for k in range(CAP // 16):
                dk = cd_v[pl.ds(k * 16, 16)]
                wk = jnp.exp(-10.0 * jnp.abs(dk))
                live = (iota + (k * 16)) < n
                cw_v[pl.ds(k * 16, 16)] = jnp.where(live, wk, 0.0)

            macc0 = zero16
            macc1 = zero16
            xacc0 = zero16
            xacc1 = zero16
            for k in range(CAP // 16):
                wv = cw_v[pl.ds(k * 16, 16)]
                iv = ci_v[pl.ds(k * 16, 16)] * P
                for l in range(16):
                    w = wv[l]
                    fo = iv[l]
                    v0 = feats_v[pl.ds(fo, 16)]
                    v1 = feats_v[pl.ds(fo + 6, 16)]
                    t0 = w * v0
                    t1 = w * v1
                    macc0 = macc0 + t0
                    macc1 = macc1 + t1
                    xacc0 = jnp.maximum(xacc0, t0)
                    xacc1 = jnp.maximum(xacc1, t1)
            ob = g * 48
            orow_v[pl.ds(ob + 32, 16)] = zero16
            orow_v[pl.ds(ob, 16)] = xacc0
            orow_v[pl.ds(ob + 6, 16)] = xacc1
            orow_v[pl.ds(ob + 22, 16)] = macc0
            orow_v[pl.ds(ob + 28, 16)] = macc1

        def pair_body(g2, carry):
            do_row(2 * g2, 0)
            do_row(2 * g2 + 1, 1)
            return carry

        lax.fori_loop(0, RPW // 2, pair_body, jnp.int32(0))
        last = jnp.minimum(row0 + RPW, NB - 1)
        pltpu.make_async_copy(dist_hbm.at[pl.ds(last * V, V)],
                              drow_v.at[pl.ds(0, V)], sem0).wait()
        pltpu.sync_copy(orow_v, out_hbm.at[pl.ds(row0 * 48, RPW * 48)])

    return sck(dist.reshape(NB * V), thr, m0, feats_flat)


def _stage2_body(x_ref, coll_ref, wo_x_ref, wo_c_ref, bo_ref, out_ref):
    xb = x_ref[0]
    cb = coll_ref[0]
    acc = _dotx(xb, wo_x_ref[...])
    acc += _dotx(cb, wo_c_ref[...])
    out_ref[0] = jnp.tanh(acc + bo_ref[...])


def _tc_stage2(x, coll, W_out, b_out):
    B, V, F = x.shape
    n_filt = W_out.shape[1]
    R = 1024
    Wo_x = W_out[:F]
    n_prop = (W_out.shape[0] - F) // 2
    Wo_c = jnp.concatenate([
        W_out[F:F + n_prop],
        W_out[F + n_prop:] / (N_NEIGHBOURS - 1.0),
        jnp.zeros((4, n_filt), jnp.float32),
    ], axis=0)
    return pl.pallas_call(
        _stage2_body,
        grid=(B, V // R),
        in_specs=[
            pl.BlockSpec((1, R, F), lambda b, j: (b, j, 0)),
            pl.BlockSpec((1, R, 48), lambda b, j: (b, j, 0)),
            pl.BlockSpec((F, n_filt), lambda b, j: (0, 0)),
            pl.BlockSpec((48, n_filt), lambda b, j: (0, 0)),
            pl.BlockSpec((1, n_filt), lambda b, j: (0, 0)),
        ],
        out_specs=pl.BlockSpec((1, R, n_filt), lambda b, j: (b, j, 0)),
        out_shape=jax.ShapeDtypeStruct((B, V, n_filt), jnp.float32),
    )(x, coll, Wo_x, Wo_c, b_out.reshape(1, n_filt))


def kernel(x, W_s, b_s, W_flr, b_flr, W_out, b_out):
    B, V, F = x.shape
    P = W_flr.shape[1]
    coords = jnp.matmul(x, W_s) + b_s
    feats = jnp.matmul(x, W_flr) + b_flr
    dist, thr, m0 = _tc_stage1(coords)
    coll = _sc_aggregate(dist, thr, m0, feats.reshape(B * V * P), B, V, P)
    return _tc_stage2(x, coll.reshape(B, V, 48), W_out, b_out)


# speedup vs baseline: 1.0264x; 1.0264x over previous
import functools

import jax
import jax.numpy as jnp
from jax import lax
from jax.experimental import pallas as pl
from jax.experimental.pallas import tpu as pltpu
from jax.experimental.pallas import tpu_sc as plsc

N_NEIGHBOURS = 40
N_BITS = 16
CAP = 48
BUF = 80


def _bf16r(a):
    bits = lax.bitcast_convert_type(a, jnp.int32)
    r = bits + jnp.int32(0x7FFF) + ((bits >> 16) & 1)
    return lax.bitcast_convert_type(r & jnp.int32(-65536), jnp.float32)


def _dot16(a, b):
    return jnp.dot(_bf16r(a), _bf16r(b), preferred_element_type=jnp.float32,
                   precision=lax.Precision.HIGHEST)


def _dotx(a, b):
    return jnp.dot(a, b, preferred_element_type=jnp.float32,
                   precision=lax.Precision.HIGHEST)


def _stage1_body(coords_ref,
                 dist_ref, thr_ref, m0_ref,
                 cT_s, cTr_s, c2r_s,
                 *, R, V, n_dim, n_prop):
    j = pl.program_id(1)
    K = N_NEIGHBOURS

    @pl.when(j == 0)
    def _precompute():
        coords = coords_ref[0]
        cT = coords.T
        cT_s[...] = cT
        cTr_s[...] = _bf16r(cT)
        c2r_s[...] = jnp.sum(cT * cT, axis=0, keepdims=True)

    rows = pl.ds(j * R, R)
    cb = coords_ref[0, rows, :]
    cbr = _bf16r(cb)
    g = cbr[:, 0:1] * cTr_s[0:1, :]
    for dd in range(1, n_dim):
        g = g + cbr[:, dd:dd + 1] * cTr_s[dd:dd + 1, :]
    c2b = jnp.sum(cb * cb, axis=1, keepdims=True)
    draw = c2b + c2r_s[...] - 2.0 * g
    dist_ref[...] = draw
    bits = lax.bitcast_convert_type(draw, jnp.int32)
    key = jnp.where(bits >= 0, bits, -(bits & jnp.int32(0x7FFFFFFF)))

    def bit_body(i, p):
        b = 30 - i
        c = p | (jnp.int32(1) << b)
        cnt = jnp.sum((key < c).astype(jnp.int32), axis=1, keepdims=True)
        return jnp.where(cnt >= K, p, c)

    p = lax.fori_loop(0, N_BITS, bit_body, jnp.zeros((R, 1), jnp.int32))
    p = p | jnp.int32((1 << (31 - N_BITS)) - 1)
    thr_ref[0] = lax.bitcast_convert_type(p, jnp.float32)
    colg = lax.broadcasted_iota(jnp.int32, (R, V), 1)
    m0val = jnp.min(key, axis=1, keepdims=True)
    m0idx = jnp.min(jnp.where(key == m0val, colg, V), axis=1, keepdims=True)
    m0_ref[0] = m0idx


def _tc_stage1(coords):
    B, V, n_dim = coords.shape
    R = 256
    NBLK = (B * V) // R
    body = functools.partial(_stage1_body, R=R, V=V, n_dim=n_dim,
                             n_prop=0)
    grid = (B, V // R)
    dist, thr, m0 = pl.pallas_call(
        body,
        grid=grid,
        in_specs=[
            pl.BlockSpec((1, V, n_dim), lambda b, j: (b, 0, 0)),
        ],
        out_specs=[
            pl.BlockSpec((R, V), lambda b, j: (b * (V // 256) + j, 0)),
            pl.BlockSpec((1, R, 1), lambda b, j: (b * (V // 256) + j, 0, 0)),
            pl.BlockSpec((1, R, 1), lambda b, j: (b * (V // 256) + j, 0, 0)),
        ],
        out_shape=[
            jax.ShapeDtypeStruct((B * V, V), jnp.float32),
            jax.ShapeDtypeStruct((NBLK, R, 1), jnp.float32),
            jax.ShapeDtypeStruct((NBLK, R, 1), jnp.int32),
        ],
        scratch_shapes=[
            pltpu.VMEM((n_dim, V), jnp.float32),
            pltpu.VMEM((n_dim, V), jnp.float32),
            pltpu.VMEM((1, V), jnp.float32),
        ],
        compiler_params=pltpu.CompilerParams(
            dimension_semantics=("arbitrary", "arbitrary"),
        ),
    )(coords)
    return dist, thr.reshape(B * V), m0.reshape(B * V)


def _sc_aggregate(dist, thr, m0, feats_flat, B, V, P):
    NB = B * V
    NW = 32
    RPW = NB // NW
    FP = V * P
    mesh = plsc.VectorSubcoreMesh(core_axis_name="c", subcore_axis_name="s")

    @functools.partial(
        pl.kernel, mesh=mesh,
        out_type=jax.ShapeDtypeStruct((NB * 48,), jnp.float32),
        scratch_types=[
            pltpu.VMEM((FP,), jnp.float32),
            pltpu.VMEM((4 * V,), jnp.float32),
            pltpu.VMEM((RPW + 16,), jnp.float32),
            pltpu.VMEM((RPW + 16,), jnp.int32),
            pltpu.VMEM((BUF,), jnp.float32),
            pltpu.VMEM((BUF,), jnp.int32),
            pltpu.VMEM((CAP,), jnp.float32),
            pltpu.VMEM((128 * 48,), jnp.float32),
            pltpu.SemaphoreType.DMA,
            pltpu.SemaphoreType.DMA,
            pltpu.SemaphoreType.DMA,
            pltpu.SemaphoreType.DMA,
        ],
        compiler_params=pltpu.CompilerParams(needs_layout_passes=False))
    def sck(dist_hbm, thr_hbm, m0_hbm, feats_hbm, out_hbm,
            feats_v, drow_v, thr_v, m0_v, cd_v, ci_v, cw_v, orow_v,
            sem0, sem1, sem2, sem3):
        wid = lax.axis_index("s") * 2 + lax.axis_index("c")
        row0 = wid * RPW
        batch = row0 // V
        i0 = row0 % V
        pltpu.sync_copy(feats_hbm.at[pl.ds(batch * FP, FP)], feats_v)
        pltpu.sync_copy(thr_hbm.at[pl.ds(row0, RPW)],
                        thr_v.at[pl.ds(0, RPW)])
        pltpu.sync_copy(m0_hbm.at[pl.ds(row0, RPW)],
                        m0_v.at[pl.ds(0, RPW)])
        iota = lax.iota(jnp.int32, 16)
        zero16 = jnp.zeros((16,), jnp.float32)
        sems = (sem0, sem1, sem2, sem3)
        for k in range(BUF // 16):
            ci_v[pl.ds(k * 16, 16)] = jnp.zeros((16,), jnp.int32)

        for pr in range(3):
            pltpu.async_copy(dist_hbm.at[pl.ds((row0 + pr) * V, V)],
                             drow_v.at[pl.ds(pr * V, V)], sems[pr])

        def do_row(g, par):
            base0 = par * V
            pltpu.make_async_copy(dist_hbm.at[pl.ds((row0 + g) * V, V)],
                                  drow_v.at[pl.ds(base0, V)],
                                  sems[par]).wait()
            nxt = jnp.minimum(row0 + g + 3, NB - 1)
            pltpu.async_copy(dist_hbm.at[pl.ds(nxt * V, V)],
                             drow_v.at[pl.ds(((par + 3) % 4) * V, V)],
                             sems[(par + 3) % 4])
            t = thr_v[pl.ds(g, 16)][0]
            i_self = m0_v[pl.ds(g, 16)][0]

            def scan_body(cc, nv):
                for u in range(8):
                    cbase = cc * 128 + u * 16
                    dv = drow_v[pl.ds(base0 + cbase, 16)]
                    jv = iota + cbase
                    m = (dv <= t) & (jv != i_self)
                    mi = jnp.where(m, jnp.int32(1), jnp.int32(0))
                    cs = plsc.cumsum(mi)
                    pc = plsc.all_reduce_population_count(m)
                    pos = nv + cs - 1
                    pos = jnp.where(m, pos, jnp.int32(BUF - 1))
                    pos = jnp.minimum(pos, jnp.int32(BUF - 1))
                    plsc.store_scatter(cd_v, [pos], dv)
                    plsc.store_scatter(ci_v, [pos], jv)
                    nv = nv + pc
                return nv

            nv = lax.fori_loop(0, V // 128, scan_body,
                               jnp.zeros((16,), jnp.int32))
            n = jnp.minimum(nv[0], CAP)

            for k in range(CAP // 16):
                dk = cd_v[pl.ds(k * 16, 16)]
                wk = jnp.exp(-10.0 * jnp.abs(dk))
                live = (iota + (k * 16)) < n
                cw_v[pl.ds(k * 16, 16)] = jnp.where(live, wk, 0.0)

            macc0 = zero16
            macc1 = zero16
            xacc0 = zero16
            xacc1 = zero16
            for k in range(CAP // 16):
                wv = cw_v[pl.ds(k * 16, 16)]
                iv = ci_v[pl.ds(k * 16, 16)] * P
                for l in range(16):
                    w = wv[l]
                    fo = iv[l]
                    v0 = feats_v[pl.ds(fo, 16)]
                    v1 = feats_v[pl.ds(fo + 6, 16)]
                    t0 = w * v0
                    t1 = w * v1
                    macc0 = macc0 + t0
                    macc1 = macc1 + t1
                    xacc0 = jnp.maximum(xacc0, t0)
                    xacc1 = jnp.maximum(xacc1, t1)
            ob = lax.rem(g, 128) * 48
            orow_v[pl.ds(ob + 32, 16)] = zero16
            orow_v[pl.ds(ob, 16)] = xacc0
            orow_v[pl.ds(ob + 6, 16)] = xacc1
            orow_v[pl.ds(ob + 22, 16)] = macc0
            orow_v[pl.ds(ob + 28, 16)] = macc1

        for blk in range(RPW // 128):
            def quad_body(g4, carry, _b=blk):
                g = _b * 128 + 4 * g4
                do_row(g, 0)
                do_row(g + 1, 1)
                do_row(g + 2, 2)
                do_row(g + 3, 3)
                return carry

            lax.fori_loop(0, 32, quad_body, jnp.int32(0))
            pltpu.sync_copy(orow_v,
                            out_hbm.at[pl.ds((row0 + blk * 128) * 48,
                                             128 * 48)])
        for pr in range(3):
            last = jnp.minimum(row0 + RPW + pr, NB - 1)
            pltpu.make_async_copy(dist_hbm.at[pl.ds(last * V, V)],
                                  drow_v.at[pl.ds(pr * V, V)],
                                  sems[pr]).wait()

    return sck(dist.reshape(NB * V), thr, m0, feats_flat)


def _stage2_body(x_ref, coll_ref, wo_x_ref, wo_c_ref, bo_ref, out_ref):
    xb = x_ref[0]
    cb = coll_ref[0]
    acc = _dotx(xb, wo_x_ref[...])
    acc += _dotx(cb, wo_c_ref[...])
    out_ref[0] = jnp.tanh(acc + bo_ref[...])


def _tc_stage2(x, coll, W_out, b_out):
    B, V, F = x.shape
    n_filt = W_out.shape[1]
    R = 1024
    Wo_x = W_out[:F]
    n_prop = (W_out.shape[0] - F) // 2
    Wo_c = jnp.concatenate([
        W_out[F:F + n_prop],
        W_out[F + n_prop:] / (N_NEIGHBOURS - 1.0),
        jnp.zeros((4, n_filt), jnp.float32),
    ], axis=0)
    return pl.pallas_call(
        _stage2_body,
        grid=(B, V // R),
        in_specs=[
            pl.BlockSpec((1, R, F), lambda b, j: (b, j, 0)),
            pl.BlockSpec((1, R, 48), lambda b, j: (b, j, 0)),
            pl.BlockSpec((F, n_filt), lambda b, j: (0, 0)),
            pl.BlockSpec((48, n_filt), lambda b, j: (0, 0)),
            pl.BlockSpec((1, n_filt), lambda b, j: (0, 0)),
        ],
        out_specs=pl.BlockSpec((1, R, n_filt), lambda b, j: (b, j, 0)),
        out_shape=jax.ShapeDtypeStruct((B, V, n_filt), jnp.float32),
    )(x, coll, Wo_x, Wo_c, b_out.reshape(1, n_filt))


def kernel(x, W_s, b_s, W_flr, b_flr, W_out, b_out):
    B, V, F = x.shape
    P = W_flr.shape[1]
    coords = jnp.matmul(x, W_s) + b_s
    feats = jnp.matmul(x, W_flr) + b_flr
    dist, thr, m0 = _tc_stage1(coords)
    coll = _sc_aggregate(dist, thr, m0, feats.reshape(B * V * P), B, V, P)
    return _tc_stage2(x, coll.reshape(B, V, 48), W_out, b_out)
